# baseline (speedup 1.0000x reference)
import jax
import jax.numpy as jnp
from jax.experimental import pallas as pl
from jax.experimental.pallas import tpu as pltpu

T = 2048
D = 1024
E = 8
DFF = 512
G = 2
NS = (E + 1 + G - 1) // G
TB = 256
NTB = T // TB


def _moe_body(x_ref, gate_ref, w1g_ref, w1u_ref, w2_ref, swg_ref, swu_ref,
              swd_ref, out_ref, wfull_ref, wgu_ref, wd_ref):
    es = pl.program_id(0)

    @pl.when(es == 0)
    def _router():
        x = x_ref[...]
        logits = jax.lax.dot_general(
            x, gate_ref[...], (((1,), (1,)), ((), ())),
            preferred_element_type=jnp.float32)
        m = jnp.max(logits, axis=-1, keepdims=True)
        ex = jnp.exp(logits - m)
        probs = ex / jnp.sum(ex, axis=-1, keepdims=True)
        lane = jax.lax.broadcasted_iota(jnp.int32, (T, E), 1)
        v1 = jnp.max(probs, axis=-1, keepdims=True)
        i1 = jnp.min(jnp.where(probs == v1, lane, E), axis=-1, keepdims=True)
        m1 = lane == i1
        probs2 = jnp.where(m1, -1.0, probs)
        v2 = jnp.max(probs2, axis=-1, keepdims=True)
        i2 = jnp.min(jnp.where(probs2 == v2, lane, E), axis=-1, keepdims=True)
        m2 = lane == i2
        denom = v1 + v2
        wfull_ref[...] = (jnp.where(m1, v1, 0.0) + jnp.where(m2, v2, 0.0)) / denom

    @pl.when(es < NS - 1)
    def _stage_routed():
        for j in range(G):
            wgu_ref[pl.ds(j * 2 * DFF, DFF), :] = w1g_ref[j].astype(jnp.bfloat16)
            wgu_ref[pl.ds(j * 2 * DFF + DFF, DFF), :] = w1u_ref[j].astype(jnp.bfloat16)
            wd_ref[:, pl.ds(j * DFF, DFF)] = w2_ref[j].astype(jnp.bfloat16)

    @pl.when(es == NS - 1)
    def _stage_shared():
        wgu_ref[pl.ds(0, DFF), :] = swg_ref[...].astype(jnp.bfloat16)
        wgu_ref[pl.ds(DFF, DFF), :] = swu_ref[...].astype(jnp.bfloat16)
        wd_ref[:, pl.ds(0, DFF)] = swd_ref[...].astype(jnp.bfloat16)

    lane = jax.lax.broadcasted_iota(jnp.int32, (TB, E), 1)
    for tb in range(NTB):
        rows = pl.ds(tb * TB, TB)
        xb = x_ref[rows, :].astype(jnp.bfloat16)
        gu = jax.lax.dot_general(xb, wgu_ref[...], (((1,), (1,)), ((), ())),
                                 preferred_element_type=jnp.float32)
        wf = wfull_ref[rows, :]
        hs = []
        for j in range(G):
            g = gu[:, j * 2 * DFF: j * 2 * DFF + DFF]
            u = gu[:, j * 2 * DFF + DFF: (j + 1) * 2 * DFF]
            eid = es * G + j
            sel = jnp.sum(jnp.where(lane == eid, wf, 0.0),
                          axis=-1, keepdims=True)
            w = jnp.where(eid == E, 1.0, sel)
            h = (g * (1.0 / (1.0 + jnp.exp(-g)))) * u * w
            hs.append(h.astype(jnp.bfloat16))
        hcat = jnp.concatenate(hs, axis=1)
        o = jax.lax.dot_general(hcat, wd_ref[...], (((1,), (1,)), ((), ())),
                                preferred_element_type=jnp.float32)

        @pl.when(es == 0)
        def _init():
            out_ref[rows, :] = o

        @pl.when(es > 0)
        def _acc():
            out_ref[rows, :] += o


@jax.jit
def kernel(hidden_states, gate_w, w1_gate, w1_up, w2, sw_gate, sw_up, sw_down):
    out = pl.pallas_call(
        _moe_body,
        grid=(NS,),
        in_specs=[
            pl.BlockSpec((T, D), lambda e: (0, 0)),
            pl.BlockSpec((E, D), lambda e: (0, 0)),
            pl.BlockSpec((G, DFF, D), lambda e: (jnp.minimum(e, E // G - 1), 0, 0)),
            pl.BlockSpec((G, DFF, D), lambda e: (jnp.minimum(e, E // G - 1), 0, 0)),
            pl.BlockSpec((G, D, DFF), lambda e: (jnp.minimum(e, E // G - 1), 0, 0)),
            pl.BlockSpec((DFF, D), lambda e: (0, 0)),
            pl.BlockSpec((DFF, D), lambda e: (0, 0)),
            pl.BlockSpec((D, DFF), lambda e: (0, 0)),
        ],
        out_specs=pl.BlockSpec((T, D), lambda e: (0, 0)),
        out_shape=jax.ShapeDtypeStruct((T, D), jnp.float32),
        scratch_shapes=[
            pltpu.VMEM((T, E), jnp.float32),
            pltpu.VMEM((G * 2 * DFF, D), jnp.bfloat16),
            pltpu.VMEM((D, G * DFF), jnp.bfloat16),
        ],
        compiler_params=pltpu.CompilerParams(
            dimension_semantics=("arbitrary",)),
    )(hidden_states, gate_w, w1_gate, w1_up, w2, sw_gate, sw_up, sw_down)
    return out


# speedup vs baseline: 1.2313x; 1.2313x over previous
import jax
import jax.numpy as jnp
from jax.experimental import pallas as pl
from jax.experimental.pallas import tpu as pltpu

T = 2048
D = 1024
E = 8
DFF = 512
TB = 512
NTB = T // TB


def _moe_body(x_ref, gate_ref, w1g_ref, w1u_ref, w2_ref, swg_ref, swu_ref,
              swd_ref, out_ref, xbf_ref, wfull_ref, wgu_ref, wd_ref):
    s = pl.program_id(0)

    @pl.when(s == 0)
    def _router():
        x = x_ref[...]
        xbf_ref[...] = x.astype(jnp.bfloat16)
        logits = jax.lax.dot_general(
            x, gate_ref[...], (((1,), (1,)), ((), ())),
            preferred_element_type=jnp.float32)
        m = jnp.max(logits, axis=-1, keepdims=True)
        ex = jnp.exp(logits - m)
        probs = ex / jnp.sum(ex, axis=-1, keepdims=True)
        lane = jax.lax.broadcasted_iota(jnp.int32, (T, E), 1)
        v1 = jnp.max(probs, axis=-1, keepdims=True)
        i1 = jnp.min(jnp.where(probs == v1, lane, E), axis=-1, keepdims=True)
        m1 = lane == i1
        probs2 = jnp.where(m1, -1.0, probs)
        v2 = jnp.max(probs2, axis=-1, keepdims=True)
        i2 = jnp.min(jnp.where(probs2 == v2, lane, E), axis=-1, keepdims=True)
        m2 = lane == i2
        denom = v1 + v2
        wfull_ref[...] = (jnp.where(m1, v1, 0.0) + jnp.where(m2, v2, 0.0)) / denom
        out_ref[...] = jnp.zeros((T, D), jnp.float32)

    buf = jax.lax.rem(s, 2)

    @pl.when(s < E)
    def _stage_routed():
        wgu_ref[buf, 0:DFF, :] = w1g_ref[0].astype(jnp.bfloat16)
        wgu_ref[buf, DFF:2 * DFF, :] = w1u_ref[0].astype(jnp.bfloat16)
        wd_ref[buf] = w2_ref[0].astype(jnp.bfloat16)

    @pl.when(s == E)
    def _stage_shared():
        wgu_ref[buf, 0:DFF, :] = swg_ref[...].astype(jnp.bfloat16)
        wgu_ref[buf, DFF:2 * DFF, :] = swu_ref[...].astype(jnp.bfloat16)
        wd_ref[buf] = swd_ref[...].astype(jnp.bfloat16)

    @pl.when(s > 0)
    def _compute():
        e = s - 1
        cbuf = 1 - buf
        lane = jax.lax.broadcasted_iota(jnp.int32, (TB, E), 1)
        for tb in range(NTB):
            rows = pl.ds(tb * TB, TB)
            xb = xbf_ref[rows, :]
            gu = jax.lax.dot_general(xb, wgu_ref[cbuf],
                                     (((1,), (1,)), ((), ())),
                                     preferred_element_type=jnp.float32)
            g = gu[:, 0:DFF]
            u = gu[:, DFF:2 * DFF]
            h = (g * (1.0 / (1.0 + jnp.exp(-g)))) * u
            o = jax.lax.dot_general(h.astype(jnp.bfloat16), wd_ref[cbuf],
                                    (((1,), (1,)), ((), ())),
                                    preferred_element_type=jnp.float32)
            sel = jnp.sum(jnp.where(lane == e, wfull_ref[rows, :], 0.0),
                          axis=-1, keepdims=True)
            w = jnp.where(e == E, 1.0, sel)
            out_ref[rows, :] += w * o


@jax.jit
def kernel(hidden_states, gate_w, w1_gate, w1_up, w2, sw_gate, sw_up, sw_down):
    grid = (E + 2,)
    out = pl.pallas_call(
        _moe_body,
        grid=grid,
        in_specs=[
            pl.BlockSpec((T, D), lambda s: (0, 0)),
            pl.BlockSpec((E, D), lambda s: (0, 0)),
            pl.BlockSpec((1, DFF, D), lambda s: (jnp.minimum(s, E - 1), 0, 0)),
            pl.BlockSpec((1, DFF, D), lambda s: (jnp.minimum(s, E - 1), 0, 0)),
            pl.BlockSpec((1, D, DFF), lambda s: (jnp.minimum(s, E - 1), 0, 0)),
            pl.BlockSpec((DFF, D), lambda s: (0, 0)),
            pl.BlockSpec((DFF, D), lambda s: (0, 0)),
            pl.BlockSpec((D, DFF), lambda s: (0, 0)),
        ],
        out_specs=pl.BlockSpec((T, D), lambda s: (0, 0)),
        out_shape=jax.ShapeDtypeStruct((T, D), jnp.float32),
        scratch_shapes=[
            pltpu.VMEM((T, D), jnp.bfloat16),
            pltpu.VMEM((T, E), jnp.float32),
            pltpu.VMEM((2, 2 * DFF, D), jnp.bfloat16),
            pltpu.VMEM((2, D, DFF), jnp.bfloat16),
        ],
        compiler_params=pltpu.CompilerParams(
            dimension_semantics=("arbitrary",)),
    )(hidden_states, gate_w, w1_gate, w1_up, w2, sw_gate, sw_up, sw_down)
    return out


# speedup vs baseline: 1.3045x; 1.0595x over previous
import jax
import jax.numpy as jnp
from jax.experimental import pallas as pl
from jax.experimental.pallas import tpu as pltpu

T = 2048
D = 1024
E = 8
DFF = 512
HC = 256
TB = 512
NTB = T // TB


def _expert_pass(wgsrc, wusrc, wdsrc, e, xbf_ref, wfull_ref, out_ref,
                 wgu_ref, wd_ref, shared):
    wgu_ref[0 * HC:1 * HC, :] = wgsrc[0:HC, :].astype(jnp.bfloat16)
    wgu_ref[1 * HC:2 * HC, :] = wusrc[0:HC, :].astype(jnp.bfloat16)
    wgu_ref[2 * HC:3 * HC, :] = wgsrc[HC:2 * HC, :].astype(jnp.bfloat16)
    wgu_ref[3 * HC:4 * HC, :] = wusrc[HC:2 * HC, :].astype(jnp.bfloat16)
    wd_ref[:, 0:HC] = wdsrc[:, 0:HC].astype(jnp.bfloat16)
    wd_ref[:, HC:2 * HC] = wdsrc[:, HC:2 * HC].astype(jnp.bfloat16)

    lane = jax.lax.broadcasted_iota(jnp.int32, (TB, E), 1)
    dn = (((1,), (1,)), ((), ()))
    for tb in range(NTB):
        rows = pl.ds(tb * TB, TB)
        xb = xbf_ref[rows, :]
        if shared:
            w = 1.0
        else:
            w = jnp.sum(jnp.where(lane == e, wfull_ref[rows, :], 0.0),
                        axis=-1, keepdims=True)
        g0 = jax.lax.dot_general(xb, wgu_ref[0 * HC:1 * HC, :], dn,
                                 preferred_element_type=jnp.float32)
        u0 = jax.lax.dot_general(xb, wgu_ref[1 * HC:2 * HC, :], dn,
                                 preferred_element_type=jnp.float32)
        h0 = ((g0 * (1.0 / (1.0 + jnp.exp(-g0)))) * u0 * w).astype(jnp.bfloat16)
        o0 = jax.lax.dot_general(h0, wd_ref[:, 0:HC], dn,
                                 preferred_element_type=jnp.float32)
        g1 = jax.lax.dot_general(xb, wgu_ref[2 * HC:3 * HC, :], dn,
                                 preferred_element_type=jnp.float32)
        u1 = jax.lax.dot_general(xb, wgu_ref[3 * HC:4 * HC, :], dn,
                                 preferred_element_type=jnp.float32)
        h1 = ((g1 * (1.0 / (1.0 + jnp.exp(-g1)))) * u1 * w).astype(jnp.bfloat16)
        o1 = jax.lax.dot_general(h1, wd_ref[:, HC:2 * HC], dn,
                                 preferred_element_type=jnp.float32)
        out_ref[rows, :] += o0 + o1


def _moe_body(x_ref, gate_ref, w1g_ref, w1u_ref, w2_ref, swg_ref, swu_ref,
              swd_ref, out_ref, xbf_ref, wfull_ref, wgu_ref, wd_ref):
    e = pl.program_id(0)

    @pl.when(e == 0)
    def _router():
        x = x_ref[...]
        xbf_ref[...] = x.astype(jnp.bfloat16)
        logits = jax.lax.dot_general(
            x, gate_ref[...], (((1,), (1,)), ((), ())),
            preferred_element_type=jnp.float32)
        m = jnp.max(logits, axis=-1, keepdims=True)
        ex = jnp.exp(logits - m)
        probs = ex / jnp.sum(ex, axis=-1, keepdims=True)
        lane = jax.lax.broadcasted_iota(jnp.int32, (T, E), 1)
        v1 = jnp.max(probs, axis=-1, keepdims=True)
        i1 = jnp.min(jnp.where(probs == v1, lane, E), axis=-1, keepdims=True)
        m1 = lane == i1
        probs2 = jnp.where(m1, -1.0, probs)
        v2 = jnp.max(probs2, axis=-1, keepdims=True)
        i2 = jnp.min(jnp.where(probs2 == v2, lane, E), axis=-1, keepdims=True)
        m2 = lane == i2
        denom = v1 + v2
        wfull_ref[...] = (jnp.where(m1, v1, 0.0) + jnp.where(m2, v2, 0.0)) / denom
        out_ref[...] = jnp.zeros((T, D), jnp.float32)

    @pl.when(e < E)
    def _routed():
        _expert_pass(w1g_ref[0], w1u_ref[0], w2_ref[0], e, xbf_ref,
                     wfull_ref, out_ref, wgu_ref, wd_ref, shared=False)

    @pl.when(e == E)
    def _shared():
        _expert_pass(swg_ref[...], swu_ref[...], swd_ref[...], e, xbf_ref,
                     wfull_ref, out_ref, wgu_ref, wd_ref, shared=True)


@jax.jit
def kernel(hidden_states, gate_w, w1_gate, w1_up, w2, sw_gate, sw_up, sw_down):
    grid = (E + 1,)
    out = pl.pallas_call(
        _moe_body,
        grid=grid,
        in_specs=[
            pl.BlockSpec((T, D), lambda e: (0, 0)),
            pl.BlockSpec((E, D), lambda e: (0, 0)),
            pl.BlockSpec((1, DFF, D), lambda e: (jnp.minimum(e, E - 1), 0, 0)),
            pl.BlockSpec((1, DFF, D), lambda e: (jnp.minimum(e, E - 1), 0, 0)),
            pl.BlockSpec((1, D, DFF), lambda e: (jnp.minimum(e, E - 1), 0, 0)),
            pl.BlockSpec((DFF, D), lambda e: (0, 0)),
            pl.BlockSpec((DFF, D), lambda e: (0, 0)),
            pl.BlockSpec((D, DFF), lambda e: (0, 0)),
        ],
        out_specs=pl.BlockSpec((T, D), lambda e: (0, 0)),
        out_shape=jax.ShapeDtypeStruct((T, D), jnp.float32),
        scratch_shapes=[
            pltpu.VMEM((T, D), jnp.bfloat16),
            pltpu.VMEM((T, E), jnp.float32),
            pltpu.VMEM((2 * DFF, D), jnp.bfloat16),
            pltpu.VMEM((D, DFF), jnp.bfloat16),
        ],
        compiler_params=pltpu.CompilerParams(
            dimension_semantics=("arbitrary",)),
    )(hidden_states, gate_w, w1_gate, w1_up, w2, sw_gate, sw_up, sw_down)
    return out
